# baseline (device time: 19864 ns/iter reference)
import jax
import jax.numpy as jnp
from jax import lax
from jax.experimental import pallas as pl
from jax.experimental.pallas import tpu as pltpu

_DeviceIdType = getattr(pl, "DeviceIdType", None) or pltpu.DeviceIdType
_MESH = _DeviceIdType.MESH
_CompilerParams = getattr(pltpu, "CompilerParams", None) or pltpu.TPUCompilerParams

_CHUNK = 512


def _chunk_schedule(half):
    sizes = [128]
    while sum(sizes) + _CHUNK <= half - 128:
        sizes.append(_CHUNK)
    rem = half - sum(sizes)
    if rem > 128:
        sizes.extend([rem - 128, 128])
    elif rem > 0:
        sizes.append(rem)
    assert sum(sizes) == half, sizes
    return sizes


def kernel(x, dy, gamma):
    m, d = x.shape
    half = m // 2
    sizes = _chunk_schedule(half)
    offs = [sum(sizes[:i]) for i in range(len(sizes))]
    n_chunks = len(sizes)

    def body(x_hbm, dy_hbm, out_ref, x_buf, dy_buf, send_ref, recv_ref,
             load_sems, send_sems, recv_sems):
        my_x = lax.axis_index("x")
        my_y = lax.axis_index("y")
        peers = [(1 - my_x, my_y), (my_x, 1 - my_y), (1 - my_x, 1 - my_y)]

        barrier = pltpu.get_barrier_semaphore()
        for p in peers:
            pl.semaphore_signal(barrier, inc=1, device_id=p,
                                device_id_type=_MESH)

        row0 = my_x * half

        def load_copies(h):
            slot = h % 2
            r = row0 + offs[h]
            sz = sizes[h]
            return (
                pltpu.make_async_copy(
                    x_hbm.at[pl.ds(r, sz), :], x_buf.at[slot, pl.ds(0, sz)],
                    load_sems.at[slot, 0]),
                pltpu.make_async_copy(
                    dy_hbm.at[pl.ds(r, sz), :], dy_buf.at[slot, pl.ds(0, sz)],
                    load_sems.at[slot, 1]),
            )

        def start_load(h):
            for cp in load_copies(h):
                cp.start()

        def wait_load(h):
            for cp in load_copies(h):
                cp.wait()

        start_load(0)
        dgamma = jnp.zeros((1, d), jnp.float32)
        dbeta = jnp.zeros((1, d), jnp.float32)
        for h in range(n_chunks):
            if h + 1 < n_chunks:
                start_load(h + 1)
            wait_load(h)
            slot = h % 2
            xv = x_buf[slot, 0:sizes[h]]
            dyv = dy_buf[slot, 0:sizes[h]]
            s1 = jnp.sum(xv, axis=1, keepdims=True)
            s2 = jnp.sum(xv * xv, axis=1, keepdims=True)
            mu = s1 * (1.0 / d)
            var = s2 * (1.0 / d) - mu * mu
            xhat = (xv - mu) * lax.rsqrt(var + 1e-5)
            dgamma = dgamma + jnp.sum(dyv * xhat, axis=0, keepdims=True)
            dbeta = dbeta + jnp.sum(dyv, axis=0, keepdims=True)
        send_ref[:, :] = jnp.concatenate([dgamma, dbeta], axis=0)

        pl.semaphore_wait(barrier, 3)

        rdmas = []
        for k, p in enumerate(peers):
            rdma = pltpu.make_async_remote_copy(
                src_ref=send_ref, dst_ref=recv_ref.at[k],
                send_sem=send_sems.at[k], recv_sem=recv_sems.at[k],
                device_id=p, device_id_type=_MESH)
            rdma.start()
            rdmas.append(rdma)
        for rdma in rdmas:
            rdma.wait()

        out_ref[:, :] = (send_ref[:, :] + recv_ref[0] + recv_ref[1]
                         + recv_ref[2])

    return pl.pallas_call(
        body,
        out_shape=jax.ShapeDtypeStruct((2, d), jnp.float32),
        in_specs=[pl.BlockSpec(memory_space=pl.ANY),
                  pl.BlockSpec(memory_space=pl.ANY)],
        out_specs=pl.BlockSpec(memory_space=pltpu.VMEM),
        scratch_shapes=[
            pltpu.VMEM((2, _CHUNK, d), jnp.float32),
            pltpu.VMEM((2, _CHUNK, d), jnp.float32),
            pltpu.VMEM((2, d), jnp.float32),
            pltpu.VMEM((3, 2, d), jnp.float32),
            pltpu.SemaphoreType.DMA((2, 2)),
            pltpu.SemaphoreType.DMA((3,)),
            pltpu.SemaphoreType.DMA((3,)),
        ],
        compiler_params=_CompilerParams(
            collective_id=0, vmem_limit_bytes=100 * 1024 * 1024),
    )(x, dy)


# device time: 19132 ns/iter; 1.0383x vs baseline; 1.0383x over previous
import jax
import jax.numpy as jnp
from jax import lax
from jax.experimental import pallas as pl
from jax.experimental.pallas import tpu as pltpu

_DeviceIdType = getattr(pl, "DeviceIdType", None) or pltpu.DeviceIdType
_MESH = _DeviceIdType.MESH
_CompilerParams = getattr(pltpu, "CompilerParams", None) or pltpu.TPUCompilerParams

_CHUNK = 256


def _chunk_schedule(half):
    sizes = [128]
    while sum(sizes) + _CHUNK <= half - 128:
        sizes.append(_CHUNK)
    rem = half - sum(sizes)
    if rem > 128:
        sizes.extend([rem - 128, 128])
    elif rem > 0:
        sizes.append(rem)
    assert sum(sizes) == half, sizes
    return sizes


def kernel(x, dy, gamma):
    m, d = x.shape
    half = m // 2
    sizes = _chunk_schedule(half)
    offs = [sum(sizes[:i]) for i in range(len(sizes))]
    n_chunks = len(sizes)

    def body(x_hbm, dy_hbm, out_ref, x_buf, dy_buf, send_ref, recv_ref,
             tot_ref, load_sems, send_sems, recv_sems, out_sem):
        my_x = lax.axis_index("x")
        my_y = lax.axis_index("y")
        peers = [(1 - my_x, my_y), (my_x, 1 - my_y), (1 - my_x, 1 - my_y)]

        barrier = pltpu.get_barrier_semaphore()
        for p in peers:
            pl.semaphore_signal(barrier, inc=1, device_id=p,
                                device_id_type=_MESH)

        row0 = my_x * half

        def load_copies(h):
            slot = h % 4
            r = row0 + offs[h]
            sz = sizes[h]
            return (
                pltpu.make_async_copy(
                    x_hbm.at[pl.ds(r, sz), :], x_buf.at[slot, pl.ds(0, sz)],
                    load_sems.at[slot, 0]),
                pltpu.make_async_copy(
                    dy_hbm.at[pl.ds(r, sz), :], dy_buf.at[slot, pl.ds(0, sz)],
                    load_sems.at[slot, 1]),
            )

        def start_load(h):
            for cp in load_copies(h):
                cp.start()

        def wait_load(h):
            for cp in load_copies(h):
                cp.wait()

        for h in range(min(3, n_chunks)):
            start_load(h)
        dgamma = jnp.zeros((1, d), jnp.float32)
        dbeta = jnp.zeros((1, d), jnp.float32)
        for h in range(n_chunks):
            if h + 3 < n_chunks:
                start_load(h + 3)
            wait_load(h)
            slot = h % 4
            xv = x_buf[slot, 0:sizes[h]]
            dyv = dy_buf[slot, 0:sizes[h]]
            s1 = jnp.sum(xv, axis=1, keepdims=True)
            s2 = jnp.sum(xv * xv, axis=1, keepdims=True)
            mu = s1 * (1.0 / d)
            var = s2 * (1.0 / d) - mu * mu
            xhat = (xv - mu) * lax.rsqrt(var + 1e-5)
            dgamma = dgamma + jnp.sum(dyv * xhat, axis=0, keepdims=True)
            dbeta = dbeta + jnp.sum(dyv, axis=0, keepdims=True)
        send_ref[:, :] = jnp.concatenate([dgamma, dbeta], axis=0)

        pl.semaphore_wait(barrier, 3)

        rdmas = []
        for k, p in enumerate(peers):
            rdma = pltpu.make_async_remote_copy(
                src_ref=send_ref, dst_ref=recv_ref.at[k],
                send_sem=send_sems.at[k], recv_sem=recv_sems.at[k],
                device_id=p, device_id_type=_MESH)
            rdma.start()
            rdmas.append(rdma)
        for rdma in rdmas:
            rdma.wait()

        tot_ref[:, :] = (send_ref[:, :] + recv_ref[0] + recv_ref[1]
                         + recv_ref[2])
        cp_out = pltpu.make_async_copy(tot_ref, out_ref, out_sem)
        cp_out.start()
        cp_out.wait()

    return pl.pallas_call(
        body,
        out_shape=jax.ShapeDtypeStruct((2, d), jnp.float32),
        in_specs=[pl.BlockSpec(memory_space=pl.ANY),
                  pl.BlockSpec(memory_space=pl.ANY)],
        out_specs=pl.BlockSpec(memory_space=pl.ANY),
        scratch_shapes=[
            pltpu.VMEM((4, _CHUNK, d), jnp.float32),
            pltpu.VMEM((4, _CHUNK, d), jnp.float32),
            pltpu.VMEM((2, d), jnp.float32),
            pltpu.VMEM((3, 2, d), jnp.float32),
            pltpu.VMEM((2, d), jnp.float32),
            pltpu.SemaphoreType.DMA((4, 2)),
            pltpu.SemaphoreType.DMA((3,)),
            pltpu.SemaphoreType.DMA((3,)),
            pltpu.SemaphoreType.DMA,
        ],
        compiler_params=_CompilerParams(
            collective_id=0, vmem_limit_bytes=100 * 1024 * 1024),
    )(x, dy)
